# baseline (speedup 1.0000x reference)
import jax
import jax.numpy as jnp
from jax.experimental import pallas as pl
from jax.experimental.pallas import tpu as pltpu

_TI = 256
_CC = 4
_SLAB = 512
_HP = jax.lax.Precision.HIGHEST


def _gat_body(x_ref, maskT_ref, wlT_ref, bl_ref, wlbT_ref, blb_ref,
              wrbT_ref, brb_ref, npos_ref, gsgn_ref, bias_ref, out_ref,
              s_ref, hlbT_ref):
    it = pl.program_id(0)
    n = x_ref.shape[0]
    cout = wlT_ref.shape[1]
    nchunks = cout // _CC
    npos = npos_ref[0, 0]

    x = x_ref[...]
    x_t = x_ref[pl.ds(it * _TI, _TI), :]
    hl = jnp.dot(x, wlT_ref[...], precision=_HP) + bl_ref[...]
    hl_t = jnp.dot(x_t, wlT_ref[...], precision=_HP) + bl_ref[...]
    hlbT = (jnp.dot(x, wlbT_ref[...], precision=_HP) + blb_ref[...]).T
    hlbT_ref[...] = hlbT
    hrb_t = jnp.dot(x_t, wrbT_ref[...], precision=_HP) + brb_ref[...]
    hlb_t = jnp.dot(x_t, wlbT_ref[...], precision=_HP) + blb_ref[...]

    sgn_l = jnp.where(
        jax.lax.broadcasted_iota(jnp.int32, (1, cout), 1) < npos, 1.0, -1.0)
    sgn_s = jnp.where(
        jax.lax.broadcasted_iota(jnp.int32, (cout, 1), 0) < npos, 1.0, -1.0)

    tb_d = hrb_t + hlb_t
    d = (1.5 * jnp.sum(sgn_l * tb_d, axis=1, keepdims=True)
         + jnp.sum(sgn_l * jnp.abs(tb_d), axis=1, keepdims=True))

    ar = 1.5 * jnp.sum(sgn_l * hrb_t, axis=1, keepdims=True)
    al_row = 1.5 * jnp.sum(sgn_s * hlbT, axis=0, keepdims=True)
    s_ref[...] = ar + al_row

    lane_iota = jax.lax.broadcasted_iota(jnp.int32, (_TI, cout), 1)

    def cbody(k, carry):
        c0 = k * _CC
        cols, gs = [], []
        for u in range(_CC):
            c = c0 + u
            gs.append(gsgn_ref[0, c])
            cols.append(jnp.sum(jnp.where(lane_iota == c, hrb_t, 0.0),
                                axis=1, keepdims=True))
        for s0 in range(0, n, _SLAB):
            acc = None
            for u in range(_CC):
                row = hlbT_ref[pl.ds(c0 + u, 1), pl.ds(s0, _SLAB)]
                term = gs[u] * jnp.abs(cols[u] + row)
                acc = term if acc is None else acc + term
            s_ref[:, pl.ds(s0, _SLAB)] += acc
        return carry

    jax.lax.fori_loop(0, nchunks, cbody, 0)

    S = s_ref[...]
    mask = maskT_ref[...] > 0
    mx = jnp.max(jnp.where(mask, S, -jnp.inf), axis=1, keepdims=True)
    mx = jnp.maximum(mx, d)
    P = jnp.where(mask, jnp.exp(S - mx), 0.0)
    p_self = jnp.exp(d - mx)
    denom = jnp.sum(P, axis=1, keepdims=True) + p_self + 1e-16
    num = jnp.dot(P, hl, precision=_HP) + p_self * hl_t
    out = num / denom + bias_ref[...]
    out_ref[...] = jnp.maximum(out, 0.0)


def _gat_layer(x, maskT, p):
    n, cin = x.shape
    cout = p["Wl"].shape[0]
    att = p["att"]
    pos = att >= 0
    order = jnp.argsort(jnp.logical_not(pos), stable=True)
    npos = jnp.sum(pos).astype(jnp.int32).reshape(1, 1)
    sa = (0.4 * jnp.abs(att))[order]
    wlT = p["Wl"].T
    bl = p["bl"].reshape(1, cout)
    wlbT = (p["Wl"][order] * sa[:, None]).T
    blb = (p["bl"][order] * sa).reshape(1, cout)
    wrbT = (p["Wr"][order] * sa[:, None]).T
    brb = (p["br"][order] * sa).reshape(1, cout)
    gsgn = jnp.where(att[order] >= 0, 1.0, -1.0).reshape(1, cout)
    bias = p["bias"].reshape(1, cout)
    return pl.pallas_call(
        _gat_body,
        grid=(n // _TI,),
        in_specs=[
            pl.BlockSpec((n, cin), lambda i: (0, 0)),
            pl.BlockSpec((_TI, n), lambda i: (i, 0)),
            pl.BlockSpec((cin, cout), lambda i: (0, 0)),
            pl.BlockSpec((1, cout), lambda i: (0, 0)),
            pl.BlockSpec((cin, cout), lambda i: (0, 0)),
            pl.BlockSpec((1, cout), lambda i: (0, 0)),
            pl.BlockSpec((cin, cout), lambda i: (0, 0)),
            pl.BlockSpec((1, cout), lambda i: (0, 0)),
            pl.BlockSpec(memory_space=pltpu.SMEM),
            pl.BlockSpec(memory_space=pltpu.SMEM),
            pl.BlockSpec((1, cout), lambda i: (0, 0)),
        ],
        out_specs=pl.BlockSpec((_TI, cout), lambda i: (i, 0)),
        out_shape=jax.ShapeDtypeStruct((n, cout), jnp.float32),
        scratch_shapes=[pltpu.VMEM((_TI, n), jnp.float32),
                        pltpu.VMEM((cout, n), jnp.float32)],
        compiler_params=pltpu.CompilerParams(
            dimension_semantics=("parallel",)),
    )(x, maskT, wlT, bl, wlbT, blb, wrbT, brb, npos, gsgn, bias)


def _recon_body(re_ref, out_ref):
    it = pl.program_id(0)
    re = re_ref[...]
    re_t = re_ref[pl.ds(it * _TI, _TI), :]
    logits = jnp.dot(re_t, re.T, precision=_HP)
    out_ref[...] = jax.nn.sigmoid(logits)


def _recon(re):
    n, c = re.shape
    return pl.pallas_call(
        _recon_body,
        grid=(n // _TI,),
        in_specs=[pl.BlockSpec((n, c), lambda i: (0, 0))],
        out_specs=pl.BlockSpec((_TI, n), lambda i: (i, 0)),
        out_shape=jax.ShapeDtypeStruct((n, n), jnp.float32),
        compiler_params=pltpu.CompilerParams(
            dimension_semantics=("parallel",)),
    )(re)


def kernel(x, edge_index, params):
    maskT = (edge_index.T != 0).astype(jnp.float32)
    x1 = _gat_layer(x, maskT, params["conv1"])
    z = _gat_layer(x1, maskT, params["conv2"])
    re = _gat_layer(z, maskT, params["edge_dec"])
    recon = _recon(re)
    xd = _gat_layer(z, maskT, params["x_dec1"])
    xr = _gat_layer(xd, maskT, params["x_dec2"])
    return recon, xr, z


# speedup vs baseline: 1.1862x; 1.1862x over previous
import jax
import jax.numpy as jnp
from jax.experimental import pallas as pl
from jax.experimental.pallas import tpu as pltpu

_TI = 256
_CC = 4
_HP = jax.lax.Precision.HIGHEST


def _lrelu(v):
    return jnp.where(v >= 0, v, 0.2 * v)


def _gat_body(x_ref, maskT_ref, wlT_ref, bl_ref, wrT_ref, br_ref, attv_ref,
              atts_ref, bias_ref, out_ref, s_ref, hlsT_ref):
    it = pl.program_id(0)
    n = x_ref.shape[0]
    cout = wlT_ref.shape[1]
    attv = attv_ref[...]
    x = x_ref[...]
    hl = jnp.dot(x, wlT_ref[...], precision=_HP) + bl_ref[...]
    hlsT_ref[...] = (hl * attv).T
    x_t = x_ref[pl.ds(it * _TI, _TI), :]
    hr_t = jnp.dot(x_t, wrT_ref[...], precision=_HP) + br_ref[...]
    hl_t = jnp.dot(x_t, wlT_ref[...], precision=_HP) + bl_ref[...]
    hrs_t = hr_t * attv

    ar = jnp.sum(hrs_t, axis=1, keepdims=True)
    al_row = jnp.sum(hlsT_ref[...], axis=0, keepdims=True)
    s_ref[...] = 0.6 * (ar + al_row)

    lane_iota = jax.lax.broadcasted_iota(jnp.int32, (_TI, cout), 1)

    def cbody(k, carry):
        acc = None
        for u in range(_CC):
            c = k * _CC + u
            a = atts_ref[0, c]
            g = jnp.where(a >= 0, jnp.float32(0.4), jnp.float32(-0.4))
            col = jnp.sum(jnp.where(lane_iota == c, hrs_t, 0.0), axis=1,
                          keepdims=True)
            row = hlsT_ref[pl.ds(c, 1), :]
            term = g * jnp.abs(col + row)
            acc = term if acc is None else acc + term
        s_ref[...] += acc
        return carry

    jax.lax.fori_loop(0, cout // _CC, cbody, 0)

    S = s_ref[...]
    mask = maskT_ref[...] > 0
    d = jnp.sum(_lrelu(hr_t + hl_t) * attv, axis=1, keepdims=True)
    mx = jnp.max(jnp.where(mask, S, -jnp.inf), axis=1, keepdims=True)
    mx = jnp.maximum(mx, d)
    P = jnp.where(mask, jnp.exp(S - mx), 0.0)
    p_self = jnp.exp(d - mx)
    denom = jnp.sum(P, axis=1, keepdims=True) + p_self + 1e-16
    num = jnp.dot(P, hl, precision=_HP) + p_self * hl_t
    out = num / denom + bias_ref[...]
    out_ref[...] = jnp.maximum(out, 0.0)


def _gat_layer(x, maskT, p):
    n, cin = x.shape
    cout = p["Wl"].shape[0]
    wlT = p["Wl"].T
    wrT = p["Wr"].T
    bl = p["bl"].reshape(1, cout)
    br = p["br"].reshape(1, cout)
    att = p["att"].reshape(1, cout)
    bias = p["bias"].reshape(1, cout)
    return pl.pallas_call(
        _gat_body,
        grid=(n // _TI,),
        in_specs=[
            pl.BlockSpec((n, cin), lambda i: (0, 0)),
            pl.BlockSpec((_TI, n), lambda i: (i, 0)),
            pl.BlockSpec((cin, cout), lambda i: (0, 0)),
            pl.BlockSpec((1, cout), lambda i: (0, 0)),
            pl.BlockSpec((cin, cout), lambda i: (0, 0)),
            pl.BlockSpec((1, cout), lambda i: (0, 0)),
            pl.BlockSpec((1, cout), lambda i: (0, 0)),
            pl.BlockSpec(memory_space=pltpu.SMEM),
            pl.BlockSpec((1, cout), lambda i: (0, 0)),
        ],
        out_specs=pl.BlockSpec((_TI, cout), lambda i: (i, 0)),
        out_shape=jax.ShapeDtypeStruct((n, cout), jnp.float32),
        scratch_shapes=[pltpu.VMEM((_TI, n), jnp.float32),
                        pltpu.VMEM((cout, n), jnp.float32)],
        compiler_params=pltpu.CompilerParams(
            dimension_semantics=("parallel",)),
    )(x, maskT, wlT, bl, wrT, br, att, att, bias)


def _recon_body(re_ref, out_ref):
    it = pl.program_id(0)
    re = re_ref[...]
    re_t = re_ref[pl.ds(it * _TI, _TI), :]
    logits = jnp.dot(re_t, re.T, precision=_HP)
    out_ref[...] = jax.nn.sigmoid(logits)


def _recon(re):
    n, c = re.shape
    return pl.pallas_call(
        _recon_body,
        grid=(n // _TI,),
        in_specs=[pl.BlockSpec((n, c), lambda i: (0, 0))],
        out_specs=pl.BlockSpec((_TI, n), lambda i: (i, 0)),
        out_shape=jax.ShapeDtypeStruct((n, n), jnp.float32),
        compiler_params=pltpu.CompilerParams(
            dimension_semantics=("parallel",)),
    )(re)


def kernel(x, edge_index, params):
    maskT = (edge_index.T != 0).astype(jnp.float32)
    x1 = _gat_layer(x, maskT, params["conv1"])
    z = _gat_layer(x1, maskT, params["conv2"])
    re = _gat_layer(z, maskT, params["edge_dec"])
    recon = _recon(re)
    xd = _gat_layer(z, maskT, params["x_dec1"])
    xr = _gat_layer(xd, maskT, params["x_dec2"])
    return recon, xr, z


# speedup vs baseline: 1.2017x; 1.0130x over previous
import jax
import jax.numpy as jnp
from jax.experimental import pallas as pl
from jax.experimental.pallas import tpu as pltpu

_TI = 256
_CC = 4
_HP = jax.lax.Precision.HIGHEST


def _lrelu(v):
    return jnp.where(v >= 0, v, 0.2 * v)


def _proj_body(x_ref, wlT_ref, bl_ref, attv_ref, hl_ref, hlsT_ref,
               alrow_ref):
    x = x_ref[...]
    hl = jnp.dot(x, wlT_ref[...], precision=_HP) + bl_ref[...]
    hl_ref[...] = hl
    hlsT = (hl * attv_ref[...]).T
    hlsT_ref[...] = hlsT
    alrow_ref[...] = jnp.sum(hlsT, axis=0, keepdims=True)


def _proj(x, wlT, bl, att):
    n, cin = x.shape
    cout = wlT.shape[1]
    return pl.pallas_call(
        _proj_body,
        in_specs=[
            pl.BlockSpec((n, cin), lambda: (0, 0)),
            pl.BlockSpec((cin, cout), lambda: (0, 0)),
            pl.BlockSpec((1, cout), lambda: (0, 0)),
            pl.BlockSpec((1, cout), lambda: (0, 0)),
        ],
        out_specs=[
            pl.BlockSpec((n, cout), lambda: (0, 0)),
            pl.BlockSpec((cout, n), lambda: (0, 0)),
            pl.BlockSpec((1, n), lambda: (0, 0)),
        ],
        out_shape=[
            jax.ShapeDtypeStruct((n, cout), jnp.float32),
            jax.ShapeDtypeStruct((cout, n), jnp.float32),
            jax.ShapeDtypeStruct((1, n), jnp.float32),
        ],
    )(x, wlT, bl, att)


def _gat_body(x_ref, maskT_ref, wrT_ref, br_ref, attv_ref,
              atts_ref, bias_ref, hl_ref, hlsT_ref, alrow_ref,
              out_ref, s_ref):
    it = pl.program_id(0)
    n = maskT_ref.shape[1]
    cout = wrT_ref.shape[1]
    attv = attv_ref[...]
    x_t = x_ref[...]
    hr_t = jnp.dot(x_t, wrT_ref[...], precision=_HP) + br_ref[...]
    hl_t = hl_ref[pl.ds(it * _TI, _TI), :]
    hrs_t = hr_t * attv

    ar = jnp.sum(hrs_t, axis=1, keepdims=True)
    s_ref[...] = 0.6 * (ar + alrow_ref[...])

    lane_iota = jax.lax.broadcasted_iota(jnp.int32, (_TI, cout), 1)

    def cbody(k, carry):
        acc = None
        for u in range(_CC):
            c = k * _CC + u
            a = atts_ref[0, c]
            g = jnp.where(a >= 0, jnp.float32(0.4), jnp.float32(-0.4))
            col = jnp.sum(jnp.where(lane_iota == c, hrs_t, 0.0), axis=1,
                          keepdims=True)
            row = hlsT_ref[pl.ds(c, 1), :]
            term = g * jnp.abs(col + row)
            acc = term if acc is None else acc + term
        s_ref[...] += acc
        return carry

    jax.lax.fori_loop(0, cout // _CC, cbody, 0)

    S = s_ref[...]
    mask = maskT_ref[...] > 0
    d = jnp.sum(_lrelu(hr_t + hl_t) * attv, axis=1, keepdims=True)
    mx = jnp.max(jnp.where(mask, S, -jnp.inf), axis=1, keepdims=True)
    mx = jnp.maximum(mx, d)
    P = jnp.where(mask, jnp.exp(S - mx), 0.0)
    p_self = jnp.exp(d - mx)
    denom = jnp.sum(P, axis=1, keepdims=True) + p_self + 1e-16
    num = jnp.dot(P, hl_ref[...], precision=_HP) + p_self * hl_t
    out = num / denom + bias_ref[...]
    out_ref[...] = jnp.maximum(out, 0.0)


def _gat_layer(x, maskT, p):
    n, cin = x.shape
    cout = p["Wl"].shape[0]
    wlT = p["Wl"].T
    wrT = p["Wr"].T
    bl = p["bl"].reshape(1, cout)
    br = p["br"].reshape(1, cout)
    att = p["att"].reshape(1, cout)
    bias = p["bias"].reshape(1, cout)
    hl, hlsT, alrow = _proj(x, wlT, bl, att)
    return pl.pallas_call(
        _gat_body,
        grid=(n // _TI,),
        in_specs=[
            pl.BlockSpec((_TI, cin), lambda i: (i, 0)),
            pl.BlockSpec((_TI, n), lambda i: (i, 0)),
            pl.BlockSpec((cin, cout), lambda i: (0, 0)),
            pl.BlockSpec((1, cout), lambda i: (0, 0)),
            pl.BlockSpec((1, cout), lambda i: (0, 0)),
            pl.BlockSpec(memory_space=pltpu.SMEM),
            pl.BlockSpec((1, cout), lambda i: (0, 0)),
            pl.BlockSpec((n, cout), lambda i: (0, 0)),
            pl.BlockSpec((cout, n), lambda i: (0, 0)),
            pl.BlockSpec((1, n), lambda i: (0, 0)),
        ],
        out_specs=pl.BlockSpec((_TI, cout), lambda i: (i, 0)),
        out_shape=jax.ShapeDtypeStruct((n, cout), jnp.float32),
        scratch_shapes=[pltpu.VMEM((_TI, n), jnp.float32)],
        compiler_params=pltpu.CompilerParams(
            dimension_semantics=("parallel",)),
    )(x, maskT, wrT, br, att, att, bias, hl, hlsT, alrow)


def _recon_body(re_ref, out_ref):
    it = pl.program_id(0)
    re = re_ref[...]
    re_t = re_ref[pl.ds(it * _TI, _TI), :]
    logits = jnp.dot(re_t, re.T, precision=_HP)
    out_ref[...] = jax.nn.sigmoid(logits)


def _recon(re):
    n, c = re.shape
    return pl.pallas_call(
        _recon_body,
        grid=(n // _TI,),
        in_specs=[pl.BlockSpec((n, c), lambda i: (0, 0))],
        out_specs=pl.BlockSpec((_TI, n), lambda i: (i, 0)),
        out_shape=jax.ShapeDtypeStruct((n, n), jnp.float32),
        compiler_params=pltpu.CompilerParams(
            dimension_semantics=("parallel",)),
    )(re)


def kernel(x, edge_index, params):
    maskT = (edge_index.T != 0).astype(jnp.float32)
    x1 = _gat_layer(x, maskT, params["conv1"])
    z = _gat_layer(x1, maskT, params["conv2"])
    re = _gat_layer(z, maskT, params["edge_dec"])
    recon = _recon(re)
    xd = _gat_layer(z, maskT, params["x_dec1"])
    xr = _gat_layer(xd, maskT, params["x_dec2"])
    return recon, xr, z


# speedup vs baseline: 1.2074x; 1.0047x over previous
import jax
import jax.numpy as jnp
from jax.experimental import pallas as pl
from jax.experimental.pallas import tpu as pltpu

_TI = 256
_CC = 4
_HP = jax.lax.Precision.HIGHEST


def _lrelu(v):
    return jnp.where(v >= 0, v, 0.2 * v)


def _proj_body(x_ref, wlT_ref, bl_ref, attv_ref, hl_ref, hlsT_ref,
               alrow_ref):
    x = x_ref[...]
    hl = jnp.dot(x, wlT_ref[...], precision=_HP) + bl_ref[...]
    hl_ref[...] = hl
    hlsT = (hl * attv_ref[...]).T
    hlsT_ref[...] = hlsT
    alrow_ref[...] = jnp.sum(hlsT, axis=0, keepdims=True)


def _proj(x, wlT, bl, att):
    n, cin = x.shape
    cout = wlT.shape[1]
    return pl.pallas_call(
        _proj_body,
        in_specs=[
            pl.BlockSpec((n, cin), lambda: (0, 0)),
            pl.BlockSpec((cin, cout), lambda: (0, 0)),
            pl.BlockSpec((1, cout), lambda: (0, 0)),
            pl.BlockSpec((1, cout), lambda: (0, 0)),
        ],
        out_specs=[
            pl.BlockSpec((n, cout), lambda: (0, 0)),
            pl.BlockSpec((cout, n), lambda: (0, 0)),
            pl.BlockSpec((1, n), lambda: (0, 0)),
        ],
        out_shape=[
            jax.ShapeDtypeStruct((n, cout), jnp.float32),
            jax.ShapeDtypeStruct((cout, n), jnp.float32),
            jax.ShapeDtypeStruct((1, n), jnp.float32),
        ],
    )(x, wlT, bl, att)


def _gat_body(x_ref, maskT_ref, wrT_ref, br_ref, attv_ref,
              atts_ref, bias_ref, hl_ref, hlsT_ref, alrow_ref,
              out_ref, s_ref):
    it = pl.program_id(0)
    n = maskT_ref.shape[1]
    cout = wrT_ref.shape[1]
    attv = attv_ref[...]
    x_t = x_ref[...]
    hr_t = jnp.dot(x_t, wrT_ref[...], precision=_HP) + br_ref[...]
    hl_t = hl_ref[pl.ds(it * _TI, _TI), :]
    hrs_t = hr_t * attv

    ar = jnp.sum(hrs_t, axis=1, keepdims=True)
    s_ref[...] = 0.6 * (ar + alrow_ref[...])

    lane_iota = jax.lax.broadcasted_iota(jnp.int32, (_TI, cout), 1)

    def cbody(k, carry):
        acc = None
        for u in range(_CC):
            c = k * _CC + u
            a = atts_ref[0, c]
            g = jnp.where(a >= 0, jnp.float32(0.4), jnp.float32(-0.4))
            col = jnp.sum(jnp.where(lane_iota == c, hrs_t, 0.0), axis=1,
                          keepdims=True)
            row = hlsT_ref[pl.ds(c, 1), :]
            term = g * jnp.abs(col + row)
            acc = term if acc is None else acc + term
        s_ref[...] += acc
        return carry

    jax.lax.fori_loop(0, cout // _CC, cbody, 0)

    Sm = s_ref[...] + maskT_ref[...]
    d = jnp.sum(_lrelu(hr_t + hl_t) * attv, axis=1, keepdims=True)
    mx = jnp.max(Sm, axis=1, keepdims=True)
    mx = jnp.maximum(mx, d)
    P = jnp.exp(Sm - mx)
    p_self = jnp.exp(d - mx)
    denom = jnp.sum(P, axis=1, keepdims=True) + p_self + 1e-16
    num = jnp.dot(P, hl_ref[...], precision=_HP) + p_self * hl_t
    out = num / denom + bias_ref[...]
    out_ref[...] = jnp.maximum(out, 0.0)


def _gat_layer(x, maskT, p):
    n, cin = x.shape
    cout = p["Wl"].shape[0]
    wlT = p["Wl"].T
    wrT = p["Wr"].T
    bl = p["bl"].reshape(1, cout)
    br = p["br"].reshape(1, cout)
    att = p["att"].reshape(1, cout)
    bias = p["bias"].reshape(1, cout)
    hl, hlsT, alrow = _proj(x, wlT, bl, att)
    return pl.pallas_call(
        _gat_body,
        grid=(n // _TI,),
        in_specs=[
            pl.BlockSpec((_TI, cin), lambda i: (i, 0)),
            pl.BlockSpec((_TI, n), lambda i: (i, 0)),
            pl.BlockSpec((cin, cout), lambda i: (0, 0)),
            pl.BlockSpec((1, cout), lambda i: (0, 0)),
            pl.BlockSpec((1, cout), lambda i: (0, 0)),
            pl.BlockSpec(memory_space=pltpu.SMEM),
            pl.BlockSpec((1, cout), lambda i: (0, 0)),
            pl.BlockSpec((n, cout), lambda i: (0, 0)),
            pl.BlockSpec((cout, n), lambda i: (0, 0)),
            pl.BlockSpec((1, n), lambda i: (0, 0)),
        ],
        out_specs=pl.BlockSpec((_TI, cout), lambda i: (i, 0)),
        out_shape=jax.ShapeDtypeStruct((n, cout), jnp.float32),
        scratch_shapes=[pltpu.VMEM((_TI, n), jnp.float32)],
        compiler_params=pltpu.CompilerParams(
            dimension_semantics=("parallel",)),
    )(x, maskT, wrT, br, att, att, bias, hl, hlsT, alrow)


def _recon_body(re_ref, out_ref):
    it = pl.program_id(0)
    re = re_ref[...]
    re_t = re_ref[pl.ds(it * _TI, _TI), :]
    logits = jnp.dot(re_t, re.T, precision=_HP)
    out_ref[...] = jax.nn.sigmoid(logits)


def _recon(re):
    n, c = re.shape
    return pl.pallas_call(
        _recon_body,
        grid=(n // _TI,),
        in_specs=[pl.BlockSpec((n, c), lambda i: (0, 0))],
        out_specs=pl.BlockSpec((_TI, n), lambda i: (i, 0)),
        out_shape=jax.ShapeDtypeStruct((n, n), jnp.float32),
        compiler_params=pltpu.CompilerParams(
            dimension_semantics=("parallel",)),
    )(re)


def kernel(x, edge_index, params):
    maskT = (edge_index.T == 0).astype(jnp.float32) * (-1e5)
    x1 = _gat_layer(x, maskT, params["conv1"])
    z = _gat_layer(x1, maskT, params["conv2"])
    re = _gat_layer(z, maskT, params["edge_dec"])
    recon = _recon(re)
    xd = _gat_layer(z, maskT, params["x_dec1"])
    xr = _gat_layer(xd, maskT, params["x_dec2"])
    return recon, xr, z


# speedup vs baseline: 1.2606x; 1.0441x over previous
import jax
import jax.numpy as jnp
from jax.experimental import pallas as pl
from jax.experimental.pallas import tpu as pltpu

_TI = 256
_CC = 4
_HP = jax.lax.Precision.HIGHEST


def _lrelu(v):
    return jnp.where(v >= 0, v, 0.2 * v)


def _proj_body(x_ref, wlT_ref, bl_ref, attv_ref, hl_ref, hlsT_ref,
               alrow_ref):
    x = x_ref[...]
    hl = jnp.dot(x, wlT_ref[...], precision=_HP) + bl_ref[...]
    hl_ref[...] = hl
    hlsT = (hl * attv_ref[...]).T
    hlsT_ref[...] = hlsT
    alrow_ref[...] = jnp.sum(hlsT, axis=0, keepdims=True)


def _proj(x, wlT, bl, att):
    n, cin = x.shape
    cout = wlT.shape[1]
    return pl.pallas_call(
        _proj_body,
        in_specs=[
            pl.BlockSpec((n, cin), lambda: (0, 0)),
            pl.BlockSpec((cin, cout), lambda: (0, 0)),
            pl.BlockSpec((1, cout), lambda: (0, 0)),
            pl.BlockSpec((1, cout), lambda: (0, 0)),
        ],
        out_specs=[
            pl.BlockSpec((n, cout), lambda: (0, 0)),
            pl.BlockSpec((cout, n), lambda: (0, 0)),
            pl.BlockSpec((1, n), lambda: (0, 0)),
        ],
        out_shape=[
            jax.ShapeDtypeStruct((n, cout), jnp.float32),
            jax.ShapeDtypeStruct((cout, n), jnp.float32),
            jax.ShapeDtypeStruct((1, n), jnp.float32),
        ],
    )(x, wlT, bl, att)


def _gat_body(x_ref, maskT_ref, wrT_ref, br_ref, attv_ref,
              atts_ref, bias_ref, hl_ref, hlsT_ref, alrow_ref,
              out_ref, s_ref):
    it = pl.program_id(0)
    n = maskT_ref.shape[1]
    cout = wrT_ref.shape[1]
    attv = attv_ref[...]
    x_t = x_ref[...]
    hr_t = jnp.dot(x_t, wrT_ref[...], precision=_HP) + br_ref[...]
    hl_t = hl_ref[pl.ds(it * _TI, _TI), :]
    hrs_t = hr_t * attv

    ar = jnp.sum(hrs_t, axis=1, keepdims=True)
    s_ref[...] = 0.6 * (ar + alrow_ref[...])

    lane_iota = jax.lax.broadcasted_iota(jnp.int32, (_TI, cout), 1)

    def cbody(k, carry):
        acc = None
        for u in range(_CC):
            c = k * _CC + u
            a = atts_ref[0, c]
            g = jnp.where(a >= 0, jnp.float32(0.4), jnp.float32(-0.4))
            col = jnp.sum(jnp.where(lane_iota == c, hrs_t, 0.0), axis=1,
                          keepdims=True)
            row = hlsT_ref[pl.ds(c, 1), :]
            term = g * jnp.abs(col + row)
            acc = term if acc is None else acc + term
        s_ref[...] += acc
        return carry

    jax.lax.fori_loop(0, cout // _CC, cbody, 0)

    Sm = s_ref[...] + maskT_ref[...]
    d = jnp.sum(_lrelu(hr_t + hl_t) * attv, axis=1, keepdims=True)
    mx = jnp.max(Sm, axis=1, keepdims=True)
    mx = jnp.maximum(mx, d)
    P = jnp.exp(Sm - mx)
    p_self = jnp.exp(d - mx)
    denom = jnp.sum(P, axis=1, keepdims=True) + p_self + 1e-16
    num = jnp.dot(P, hl_ref[...],
                  precision=jax.lax.Precision.DEFAULT) + p_self * hl_t
    out = num / denom + bias_ref[...]
    out_ref[...] = jnp.maximum(out, 0.0)


def _gat_layer(x, maskT, p):
    n, cin = x.shape
    cout = p["Wl"].shape[0]
    wlT = p["Wl"].T
    wrT = p["Wr"].T
    bl = p["bl"].reshape(1, cout)
    br = p["br"].reshape(1, cout)
    att = p["att"].reshape(1, cout)
    bias = p["bias"].reshape(1, cout)
    hl, hlsT, alrow = _proj(x, wlT, bl, att)
    return pl.pallas_call(
        _gat_body,
        grid=(n // _TI,),
        in_specs=[
            pl.BlockSpec((_TI, cin), lambda i: (i, 0)),
            pl.BlockSpec((_TI, n), lambda i: (i, 0)),
            pl.BlockSpec((cin, cout), lambda i: (0, 0)),
            pl.BlockSpec((1, cout), lambda i: (0, 0)),
            pl.BlockSpec((1, cout), lambda i: (0, 0)),
            pl.BlockSpec(memory_space=pltpu.SMEM),
            pl.BlockSpec((1, cout), lambda i: (0, 0)),
            pl.BlockSpec((n, cout), lambda i: (0, 0)),
            pl.BlockSpec((cout, n), lambda i: (0, 0)),
            pl.BlockSpec((1, n), lambda i: (0, 0)),
        ],
        out_specs=pl.BlockSpec((_TI, cout), lambda i: (i, 0)),
        out_shape=jax.ShapeDtypeStruct((n, cout), jnp.float32),
        scratch_shapes=[pltpu.VMEM((_TI, n), jnp.float32)],
        compiler_params=pltpu.CompilerParams(
            dimension_semantics=("parallel",)),
    )(x, maskT, wrT, br, att, att, bias, hl, hlsT, alrow)


def _recon_body(re_ref, out_ref):
    it = pl.program_id(0)
    re = re_ref[...]
    re_t = re_ref[pl.ds(it * _TI, _TI), :]
    logits = jnp.dot(re_t, re.T, precision=_HP)
    out_ref[...] = jax.nn.sigmoid(logits)


def _recon(re):
    n, c = re.shape
    return pl.pallas_call(
        _recon_body,
        grid=(n // _TI,),
        in_specs=[pl.BlockSpec((n, c), lambda i: (0, 0))],
        out_specs=pl.BlockSpec((_TI, n), lambda i: (i, 0)),
        out_shape=jax.ShapeDtypeStruct((n, n), jnp.float32),
        compiler_params=pltpu.CompilerParams(
            dimension_semantics=("parallel",)),
    )(re)


def kernel(x, edge_index, params):
    maskT = (edge_index.T == 0).astype(jnp.float32) * (-1e5)
    x1 = _gat_layer(x, maskT, params["conv1"])
    z = _gat_layer(x1, maskT, params["conv2"])
    re = _gat_layer(z, maskT, params["edge_dec"])
    recon = _recon(re)
    xd = _gat_layer(z, maskT, params["x_dec1"])
    xr = _gat_layer(xd, maskT, params["x_dec2"])
    return recon, xr, z


# speedup vs baseline: 1.3484x; 1.0696x over previous
import jax
import jax.numpy as jnp
from jax.experimental import pallas as pl
from jax.experimental.pallas import tpu as pltpu

_TI = 256
_CC = 8
_HP = jax.lax.Precision.HIGHEST


def _lrelu(v):
    return jnp.where(v >= 0, v, 0.2 * v)


def _proj_body(x_ref, wlT_ref, bl_ref, attv_ref, hl_ref, hlsT_ref,
               alrow_ref):
    x = x_ref[...]
    hl = jnp.dot(x, wlT_ref[...], precision=_HP) + bl_ref[...]
    hl_ref[...] = hl
    hlsT = (hl * attv_ref[...]).T
    hlsT_ref[...] = hlsT
    alrow_ref[...] = jnp.sum(hlsT, axis=0, keepdims=True)


def _proj(x, wlT, bl, att):
    n, cin = x.shape
    cout = wlT.shape[1]
    return pl.pallas_call(
        _proj_body,
        in_specs=[
            pl.BlockSpec((n, cin), lambda: (0, 0)),
            pl.BlockSpec((cin, cout), lambda: (0, 0)),
            pl.BlockSpec((1, cout), lambda: (0, 0)),
            pl.BlockSpec((1, cout), lambda: (0, 0)),
        ],
        out_specs=[
            pl.BlockSpec((n, cout), lambda: (0, 0)),
            pl.BlockSpec((cout, n), lambda: (0, 0)),
            pl.BlockSpec((1, n), lambda: (0, 0)),
        ],
        out_shape=[
            jax.ShapeDtypeStruct((n, cout), jnp.float32),
            jax.ShapeDtypeStruct((cout, n), jnp.float32),
            jax.ShapeDtypeStruct((1, n), jnp.float32),
        ],
    )(x, wlT, bl, att)


def _gat_body(x_ref, maskT_ref, wrT_ref, br_ref, attv_ref,
              atts_ref, bias_ref, hl_ref, hlsT_ref, alrow_ref,
              out_ref, s_ref):
    it = pl.program_id(0)
    n = maskT_ref.shape[1]
    cout = wrT_ref.shape[1]
    attv = attv_ref[...]
    x_t = x_ref[...]
    hr_t = jnp.dot(x_t, wrT_ref[...], precision=_HP) + br_ref[...]
    hl_t = hl_ref[pl.ds(it * _TI, _TI), :]
    hrs_t = hr_t * attv

    ar = jnp.sum(hrs_t, axis=1, keepdims=True)
    s_ref[...] = 0.6 * (ar + alrow_ref[...])

    lane_iota = jax.lax.broadcasted_iota(jnp.int32, (_TI, cout), 1)

    def cbody(k, carry):
        acc = None
        for u in range(_CC):
            c = k * _CC + u
            a = atts_ref[0, c]
            g = jnp.where(a >= 0, jnp.float32(0.4), jnp.float32(-0.4))
            col = jnp.sum(jnp.where(lane_iota == c, hrs_t, 0.0), axis=1,
                          keepdims=True)
            row = hlsT_ref[pl.ds(c, 1), :]
            term = g * jnp.abs(col + row)
            acc = term if acc is None else acc + term
        s_ref[...] += acc
        return carry

    jax.lax.fori_loop(0, cout // _CC, cbody, 0)

    Sm = s_ref[...] + maskT_ref[...]
    d = jnp.sum(_lrelu(hr_t + hl_t) * attv, axis=1, keepdims=True)
    mx = jnp.max(Sm, axis=1, keepdims=True)
    mx = jnp.maximum(mx, d)
    P = jnp.exp(Sm - mx)
    p_self = jnp.exp(d - mx)
    denom = jnp.sum(P, axis=1, keepdims=True) + p_self + 1e-16
    num = jnp.dot(P, hl_ref[...],
                  precision=jax.lax.Precision.DEFAULT) + p_self * hl_t
    out = num / denom + bias_ref[...]
    out_ref[...] = jnp.maximum(out, 0.0)


def _gat_layer(x, maskT, p):
    n, cin = x.shape
    cout = p["Wl"].shape[0]
    wlT = p["Wl"].T
    wrT = p["Wr"].T
    bl = p["bl"].reshape(1, cout)
    br = p["br"].reshape(1, cout)
    att = p["att"].reshape(1, cout)
    bias = p["bias"].reshape(1, cout)
    hl, hlsT, alrow = _proj(x, wlT, bl, att)
    return pl.pallas_call(
        _gat_body,
        grid=(n // _TI,),
        in_specs=[
            pl.BlockSpec((_TI, cin), lambda i: (i, 0)),
            pl.BlockSpec((_TI, n), lambda i: (i, 0)),
            pl.BlockSpec((cin, cout), lambda i: (0, 0)),
            pl.BlockSpec((1, cout), lambda i: (0, 0)),
            pl.BlockSpec((1, cout), lambda i: (0, 0)),
            pl.BlockSpec(memory_space=pltpu.SMEM),
            pl.BlockSpec((1, cout), lambda i: (0, 0)),
            pl.BlockSpec((n, cout), lambda i: (0, 0)),
            pl.BlockSpec((cout, n), lambda i: (0, 0)),
            pl.BlockSpec((1, n), lambda i: (0, 0)),
        ],
        out_specs=pl.BlockSpec((_TI, cout), lambda i: (i, 0)),
        out_shape=jax.ShapeDtypeStruct((n, cout), jnp.float32),
        scratch_shapes=[pltpu.VMEM((_TI, n), jnp.float32)],
        compiler_params=pltpu.CompilerParams(
            dimension_semantics=("parallel",)),
    )(x, maskT, wrT, br, att, att, bias, hl, hlsT, alrow)


def _recon_body(re_ref, out_ref):
    it = pl.program_id(0)
    re = re_ref[...]
    re_t = re_ref[pl.ds(it * _TI, _TI), :]
    logits = jnp.dot(re_t, re.T, precision=_HP)
    out_ref[...] = jax.nn.sigmoid(logits)


def _recon(re):
    n, c = re.shape
    return pl.pallas_call(
        _recon_body,
        grid=(n // _TI,),
        in_specs=[pl.BlockSpec((n, c), lambda i: (0, 0))],
        out_specs=pl.BlockSpec((_TI, n), lambda i: (i, 0)),
        out_shape=jax.ShapeDtypeStruct((n, n), jnp.float32),
        compiler_params=pltpu.CompilerParams(
            dimension_semantics=("parallel",)),
    )(re)


def kernel(x, edge_index, params):
    maskT = (edge_index.T == 0).astype(jnp.float32) * (-1e5)
    x1 = _gat_layer(x, maskT, params["conv1"])
    z = _gat_layer(x1, maskT, params["conv2"])
    re = _gat_layer(z, maskT, params["edge_dec"])
    recon = _recon(re)
    xd = _gat_layer(z, maskT, params["x_dec1"])
    xr = _gat_layer(xd, maskT, params["x_dec2"])
    return recon, xr, z


# speedup vs baseline: 1.3723x; 1.0177x over previous
import jax
import jax.numpy as jnp
from jax.experimental import pallas as pl
from jax.experimental.pallas import tpu as pltpu

_TI = 256
_CC = 16
_HP = jax.lax.Precision.HIGHEST


def _lrelu(v):
    return jnp.where(v >= 0, v, 0.2 * v)


def _proj_body(x_ref, wlT_ref, bl_ref, attv_ref, hl_ref, hlsT_ref,
               alrow_ref):
    x = x_ref[...]
    hl = jnp.dot(x, wlT_ref[...], precision=_HP) + bl_ref[...]
    hl_ref[...] = hl
    hlsT = (hl * attv_ref[...]).T
    hlsT_ref[...] = hlsT
    alrow_ref[...] = jnp.sum(hlsT, axis=0, keepdims=True)


def _proj(x, wlT, bl, att):
    n, cin = x.shape
    cout = wlT.shape[1]
    return pl.pallas_call(
        _proj_body,
        in_specs=[
            pl.BlockSpec((n, cin), lambda: (0, 0)),
            pl.BlockSpec((cin, cout), lambda: (0, 0)),
            pl.BlockSpec((1, cout), lambda: (0, 0)),
            pl.BlockSpec((1, cout), lambda: (0, 0)),
        ],
        out_specs=[
            pl.BlockSpec((n, cout), lambda: (0, 0)),
            pl.BlockSpec((cout, n), lambda: (0, 0)),
            pl.BlockSpec((1, n), lambda: (0, 0)),
        ],
        out_shape=[
            jax.ShapeDtypeStruct((n, cout), jnp.float32),
            jax.ShapeDtypeStruct((cout, n), jnp.float32),
            jax.ShapeDtypeStruct((1, n), jnp.float32),
        ],
    )(x, wlT, bl, att)


def _gat_body(x_ref, maskT_ref, wrT_ref, br_ref, attv_ref,
              atts_ref, bias_ref, hl_ref, hlsT_ref, alrow_ref,
              out_ref, s_ref):
    it = pl.program_id(0)
    n = maskT_ref.shape[1]
    cout = wrT_ref.shape[1]
    attv = attv_ref[...]
    x_t = x_ref[...]
    hr_t = jnp.dot(x_t, wrT_ref[...], precision=_HP) + br_ref[...]
    hl_t = hl_ref[pl.ds(it * _TI, _TI), :]
    hrs_t = hr_t * attv

    ar = jnp.sum(hrs_t, axis=1, keepdims=True)
    s_ref[...] = 0.6 * (ar + alrow_ref[...])

    lane_iota = jax.lax.broadcasted_iota(jnp.int32, (_TI, cout), 1)

    def cbody(k, carry):
        acc = None
        for u in range(_CC):
            c = k * _CC + u
            a = atts_ref[0, c]
            g = jnp.where(a >= 0, jnp.float32(0.4), jnp.float32(-0.4))
            col = jnp.sum(jnp.where(lane_iota == c, hrs_t, 0.0), axis=1,
                          keepdims=True)
            row = hlsT_ref[pl.ds(c, 1), :]
            term = g * jnp.abs(col + row)
            acc = term if acc is None else acc + term
        s_ref[...] += acc
        return carry

    jax.lax.fori_loop(0, cout // _CC, cbody, 0)

    Sm = s_ref[...] + maskT_ref[...]
    d = jnp.sum(_lrelu(hr_t + hl_t) * attv, axis=1, keepdims=True)
    mx = jnp.max(Sm, axis=1, keepdims=True)
    mx = jnp.maximum(mx, d)
    P = jnp.exp(Sm - mx)
    p_self = jnp.exp(d - mx)
    denom = jnp.sum(P, axis=1, keepdims=True) + p_self + 1e-16
    num = jnp.dot(P, hl_ref[...],
                  precision=jax.lax.Precision.DEFAULT) + p_self * hl_t
    out = num / denom + bias_ref[...]
    out_ref[...] = jnp.maximum(out, 0.0)


def _gat_layer(x, maskT, p):
    n, cin = x.shape
    cout = p["Wl"].shape[0]
    wlT = p["Wl"].T
    wrT = p["Wr"].T
    bl = p["bl"].reshape(1, cout)
    br = p["br"].reshape(1, cout)
    att = p["att"].reshape(1, cout)
    bias = p["bias"].reshape(1, cout)
    hl, hlsT, alrow = _proj(x, wlT, bl, att)
    return pl.pallas_call(
        _gat_body,
        grid=(n // _TI,),
        in_specs=[
            pl.BlockSpec((_TI, cin), lambda i: (i, 0)),
            pl.BlockSpec((_TI, n), lambda i: (i, 0)),
            pl.BlockSpec((cin, cout), lambda i: (0, 0)),
            pl.BlockSpec((1, cout), lambda i: (0, 0)),
            pl.BlockSpec((1, cout), lambda i: (0, 0)),
            pl.BlockSpec(memory_space=pltpu.SMEM),
            pl.BlockSpec((1, cout), lambda i: (0, 0)),
            pl.BlockSpec((n, cout), lambda i: (0, 0)),
            pl.BlockSpec((cout, n), lambda i: (0, 0)),
            pl.BlockSpec((1, n), lambda i: (0, 0)),
        ],
        out_specs=pl.BlockSpec((_TI, cout), lambda i: (i, 0)),
        out_shape=jax.ShapeDtypeStruct((n, cout), jnp.float32),
        scratch_shapes=[pltpu.VMEM((_TI, n), jnp.float32)],
        compiler_params=pltpu.CompilerParams(
            dimension_semantics=("parallel",)),
    )(x, maskT, wrT, br, att, att, bias, hl, hlsT, alrow)


def _recon_body(re_ref, out_ref):
    it = pl.program_id(0)
    re = re_ref[...]
    re_t = re_ref[pl.ds(it * _TI, _TI), :]
    logits = jnp.dot(re_t, re.T, precision=_HP)
    out_ref[...] = jax.nn.sigmoid(logits)


def _recon(re):
    n, c = re.shape
    return pl.pallas_call(
        _recon_body,
        grid=(n // _TI,),
        in_specs=[pl.BlockSpec((n, c), lambda i: (0, 0))],
        out_specs=pl.BlockSpec((_TI, n), lambda i: (i, 0)),
        out_shape=jax.ShapeDtypeStruct((n, n), jnp.float32),
        compiler_params=pltpu.CompilerParams(
            dimension_semantics=("parallel",)),
    )(re)


def kernel(x, edge_index, params):
    maskT = (edge_index.T == 0).astype(jnp.float32) * (-1e5)
    x1 = _gat_layer(x, maskT, params["conv1"])
    z = _gat_layer(x1, maskT, params["conv2"])
    re = _gat_layer(z, maskT, params["edge_dec"])
    recon = _recon(re)
    xd = _gat_layer(z, maskT, params["x_dec1"])
    xr = _gat_layer(xd, maskT, params["x_dec2"])
    return recon, xr, z


# speedup vs baseline: 1.3841x; 1.0086x over previous
import jax
import jax.numpy as jnp
from jax.experimental import pallas as pl
from jax.experimental.pallas import tpu as pltpu

_TI = 256
_CC = 32
_HP = jax.lax.Precision.HIGHEST


def _lrelu(v):
    return jnp.where(v >= 0, v, 0.2 * v)


def _proj_body(x_ref, wlT_ref, bl_ref, attv_ref, hl_ref, hlsT_ref,
               alrow_ref):
    x = x_ref[...]
    hl = jnp.dot(x, wlT_ref[...], precision=_HP) + bl_ref[...]
    hl_ref[...] = hl
    hlsT = (hl * attv_ref[...]).T
    hlsT_ref[...] = hlsT
    alrow_ref[...] = jnp.sum(hlsT, axis=0, keepdims=True)


def _proj(x, wlT, bl, att):
    n, cin = x.shape
    cout = wlT.shape[1]
    return pl.pallas_call(
        _proj_body,
        in_specs=[
            pl.BlockSpec((n, cin), lambda: (0, 0)),
            pl.BlockSpec((cin, cout), lambda: (0, 0)),
            pl.BlockSpec((1, cout), lambda: (0, 0)),
            pl.BlockSpec((1, cout), lambda: (0, 0)),
        ],
        out_specs=[
            pl.BlockSpec((n, cout), lambda: (0, 0)),
            pl.BlockSpec((cout, n), lambda: (0, 0)),
            pl.BlockSpec((1, n), lambda: (0, 0)),
        ],
        out_shape=[
            jax.ShapeDtypeStruct((n, cout), jnp.float32),
            jax.ShapeDtypeStruct((cout, n), jnp.float32),
            jax.ShapeDtypeStruct((1, n), jnp.float32),
        ],
    )(x, wlT, bl, att)


def _gat_body(x_ref, maskT_ref, wrT_ref, br_ref, attv_ref,
              atts_ref, bias_ref, hl_ref, hlsT_ref, alrow_ref,
              out_ref, s_ref):
    it = pl.program_id(0)
    n = maskT_ref.shape[1]
    cout = wrT_ref.shape[1]
    attv = attv_ref[...]
    x_t = x_ref[...]
    hr_t = jnp.dot(x_t, wrT_ref[...], precision=_HP) + br_ref[...]
    hl_t = hl_ref[pl.ds(it * _TI, _TI), :]
    hrs_t = hr_t * attv

    ar = jnp.sum(hrs_t, axis=1, keepdims=True)
    s_ref[...] = 0.6 * (ar + alrow_ref[...])

    lane_iota = jax.lax.broadcasted_iota(jnp.int32, (_TI, cout), 1)

    def cbody(k, carry):
        acc = None
        for u in range(_CC):
            c = k * _CC + u
            a = atts_ref[0, c]
            g = jnp.where(a >= 0, jnp.float32(0.4), jnp.float32(-0.4))
            col = jnp.sum(jnp.where(lane_iota == c, hrs_t, 0.0), axis=1,
                          keepdims=True)
            row = hlsT_ref[pl.ds(c, 1), :]
            term = g * jnp.abs(col + row)
            acc = term if acc is None else acc + term
        s_ref[...] += acc
        return carry

    jax.lax.fori_loop(0, cout // _CC, cbody, 0)

    Sm = s_ref[...] + maskT_ref[...]
    d = jnp.sum(_lrelu(hr_t + hl_t) * attv, axis=1, keepdims=True)
    mx = jnp.max(Sm, axis=1, keepdims=True)
    mx = jnp.maximum(mx, d)
    P = jnp.exp(Sm - mx)
    p_self = jnp.exp(d - mx)
    denom = jnp.sum(P, axis=1, keepdims=True) + p_self + 1e-16
    num = jnp.dot(P, hl_ref[...],
                  precision=jax.lax.Precision.DEFAULT) + p_self * hl_t
    out = num / denom + bias_ref[...]
    out_ref[...] = jnp.maximum(out, 0.0)


def _gat_layer(x, maskT, p):
    n, cin = x.shape
    cout = p["Wl"].shape[0]
    wlT = p["Wl"].T
    wrT = p["Wr"].T
    bl = p["bl"].reshape(1, cout)
    br = p["br"].reshape(1, cout)
    att = p["att"].reshape(1, cout)
    bias = p["bias"].reshape(1, cout)
    hl, hlsT, alrow = _proj(x, wlT, bl, att)
    return pl.pallas_call(
        _gat_body,
        grid=(n // _TI,),
        in_specs=[
            pl.BlockSpec((_TI, cin), lambda i: (i, 0)),
            pl.BlockSpec((_TI, n), lambda i: (i, 0)),
            pl.BlockSpec((cin, cout), lambda i: (0, 0)),
            pl.BlockSpec((1, cout), lambda i: (0, 0)),
            pl.BlockSpec((1, cout), lambda i: (0, 0)),
            pl.BlockSpec(memory_space=pltpu.SMEM),
            pl.BlockSpec((1, cout), lambda i: (0, 0)),
            pl.BlockSpec((n, cout), lambda i: (0, 0)),
            pl.BlockSpec((cout, n), lambda i: (0, 0)),
            pl.BlockSpec((1, n), lambda i: (0, 0)),
        ],
        out_specs=pl.BlockSpec((_TI, cout), lambda i: (i, 0)),
        out_shape=jax.ShapeDtypeStruct((n, cout), jnp.float32),
        scratch_shapes=[pltpu.VMEM((_TI, n), jnp.float32)],
        compiler_params=pltpu.CompilerParams(
            dimension_semantics=("parallel",)),
    )(x, maskT, wrT, br, att, att, bias, hl, hlsT, alrow)


def _recon_body(re_ref, out_ref):
    it = pl.program_id(0)
    re = re_ref[...]
    re_t = re_ref[pl.ds(it * _TI, _TI), :]
    logits = jnp.dot(re_t, re.T, precision=_HP)
    out_ref[...] = jax.nn.sigmoid(logits)


def _recon(re):
    n, c = re.shape
    return pl.pallas_call(
        _recon_body,
        grid=(n // _TI,),
        in_specs=[pl.BlockSpec((n, c), lambda i: (0, 0))],
        out_specs=pl.BlockSpec((_TI, n), lambda i: (i, 0)),
        out_shape=jax.ShapeDtypeStruct((n, n), jnp.float32),
        compiler_params=pltpu.CompilerParams(
            dimension_semantics=("parallel",)),
    )(re)


def kernel(x, edge_index, params):
    maskT = (edge_index.T == 0).astype(jnp.float32) * (-1e5)
    x1 = _gat_layer(x, maskT, params["conv1"])
    z = _gat_layer(x1, maskT, params["conv2"])
    re = _gat_layer(z, maskT, params["edge_dec"])
    recon = _recon(re)
    xd = _gat_layer(z, maskT, params["x_dec1"])
    xr = _gat_layer(xd, maskT, params["x_dec2"])
    return recon, xr, z


# speedup vs baseline: 1.3927x; 1.0062x over previous
import jax
import jax.numpy as jnp
from jax.experimental import pallas as pl
from jax.experimental.pallas import tpu as pltpu

_TI = 256
_CC = 64
_HP = jax.lax.Precision.HIGHEST


def _lrelu(v):
    return jnp.where(v >= 0, v, 0.2 * v)


def _proj_body(x_ref, wlT_ref, bl_ref, attv_ref, hl_ref, hlsT_ref,
               alrow_ref):
    x = x_ref[...]
    hl = jnp.dot(x, wlT_ref[...], precision=_HP) + bl_ref[...]
    hl_ref[...] = hl
    hlsT = (hl * attv_ref[...]).T
    hlsT_ref[...] = hlsT
    alrow_ref[...] = jnp.sum(hlsT, axis=0, keepdims=True)


def _proj(x, wlT, bl, att):
    n, cin = x.shape
    cout = wlT.shape[1]
    return pl.pallas_call(
        _proj_body,
        in_specs=[
            pl.BlockSpec((n, cin), lambda: (0, 0)),
            pl.BlockSpec((cin, cout), lambda: (0, 0)),
            pl.BlockSpec((1, cout), lambda: (0, 0)),
            pl.BlockSpec((1, cout), lambda: (0, 0)),
        ],
        out_specs=[
            pl.BlockSpec((n, cout), lambda: (0, 0)),
            pl.BlockSpec((cout, n), lambda: (0, 0)),
            pl.BlockSpec((1, n), lambda: (0, 0)),
        ],
        out_shape=[
            jax.ShapeDtypeStruct((n, cout), jnp.float32),
            jax.ShapeDtypeStruct((cout, n), jnp.float32),
            jax.ShapeDtypeStruct((1, n), jnp.float32),
        ],
    )(x, wlT, bl, att)


def _gat_body(x_ref, maskT_ref, wrT_ref, br_ref, attv_ref,
              atts_ref, bias_ref, hl_ref, hlsT_ref, alrow_ref,
              out_ref, s_ref):
    it = pl.program_id(0)
    n = maskT_ref.shape[1]
    cout = wrT_ref.shape[1]
    attv = attv_ref[...]
    x_t = x_ref[...]
    hr_t = jnp.dot(x_t, wrT_ref[...], precision=_HP) + br_ref[...]
    hl_t = hl_ref[pl.ds(it * _TI, _TI), :]
    hrs_t = hr_t * attv

    ar = jnp.sum(hrs_t, axis=1, keepdims=True)
    s_ref[...] = 0.6 * (ar + alrow_ref[...])

    lane_iota = jax.lax.broadcasted_iota(jnp.int32, (_TI, cout), 1)

    def cbody(k, carry):
        acc = None
        for u in range(_CC):
            c = k * _CC + u
            a = atts_ref[0, c]
            g = jnp.where(a >= 0, jnp.float32(0.4), jnp.float32(-0.4))
            col = jnp.sum(jnp.where(lane_iota == c, hrs_t, 0.0), axis=1,
                          keepdims=True)
            row = hlsT_ref[pl.ds(c, 1), :]
            term = g * jnp.abs(col + row)
            acc = term if acc is None else acc + term
        s_ref[...] += acc
        return carry

    jax.lax.fori_loop(0, cout // _CC, cbody, 0)

    Sm = s_ref[...] + maskT_ref[...]
    d = jnp.sum(_lrelu(hr_t + hl_t) * attv, axis=1, keepdims=True)
    mx = jnp.max(Sm, axis=1, keepdims=True)
    mx = jnp.maximum(mx, d)
    P = jnp.exp(Sm - mx)
    p_self = jnp.exp(d - mx)
    denom = jnp.sum(P, axis=1, keepdims=True) + p_self + 1e-16
    num = jnp.dot(P, hl_ref[...],
                  precision=jax.lax.Precision.DEFAULT) + p_self * hl_t
    out = num / denom + bias_ref[...]
    out_ref[...] = jnp.maximum(out, 0.0)


def _gat_layer(x, maskT, p):
    n, cin = x.shape
    cout = p["Wl"].shape[0]
    wlT = p["Wl"].T
    wrT = p["Wr"].T
    bl = p["bl"].reshape(1, cout)
    br = p["br"].reshape(1, cout)
    att = p["att"].reshape(1, cout)
    bias = p["bias"].reshape(1, cout)
    hl, hlsT, alrow = _proj(x, wlT, bl, att)
    return pl.pallas_call(
        _gat_body,
        grid=(n // _TI,),
        in_specs=[
            pl.BlockSpec((_TI, cin), lambda i: (i, 0)),
            pl.BlockSpec((_TI, n), lambda i: (i, 0)),
            pl.BlockSpec((cin, cout), lambda i: (0, 0)),
            pl.BlockSpec((1, cout), lambda i: (0, 0)),
            pl.BlockSpec((1, cout), lambda i: (0, 0)),
            pl.BlockSpec(memory_space=pltpu.SMEM),
            pl.BlockSpec((1, cout), lambda i: (0, 0)),
            pl.BlockSpec((n, cout), lambda i: (0, 0)),
            pl.BlockSpec((cout, n), lambda i: (0, 0)),
            pl.BlockSpec((1, n), lambda i: (0, 0)),
        ],
        out_specs=pl.BlockSpec((_TI, cout), lambda i: (i, 0)),
        out_shape=jax.ShapeDtypeStruct((n, cout), jnp.float32),
        scratch_shapes=[pltpu.VMEM((_TI, n), jnp.float32)],
        compiler_params=pltpu.CompilerParams(
            dimension_semantics=("parallel",)),
    )(x, maskT, wrT, br, att, att, bias, hl, hlsT, alrow)


def _recon_body(re_ref, out_ref):
    it = pl.program_id(0)
    re = re_ref[...]
    re_t = re_ref[pl.ds(it * _TI, _TI), :]
    logits = jnp.dot(re_t, re.T, precision=_HP)
    out_ref[...] = jax.nn.sigmoid(logits)


def _recon(re):
    n, c = re.shape
    return pl.pallas_call(
        _recon_body,
        grid=(n // _TI,),
        in_specs=[pl.BlockSpec((n, c), lambda i: (0, 0))],
        out_specs=pl.BlockSpec((_TI, n), lambda i: (i, 0)),
        out_shape=jax.ShapeDtypeStruct((n, n), jnp.float32),
        compiler_params=pltpu.CompilerParams(
            dimension_semantics=("parallel",)),
    )(re)


def kernel(x, edge_index, params):
    maskT = (edge_index.T == 0).astype(jnp.float32) * (-1e5)
    x1 = _gat_layer(x, maskT, params["conv1"])
    z = _gat_layer(x1, maskT, params["conv2"])
    re = _gat_layer(z, maskT, params["edge_dec"])
    recon = _recon(re)
    xd = _gat_layer(z, maskT, params["x_dec1"])
    xr = _gat_layer(xd, maskT, params["x_dec2"])
    return recon, xr, z


# speedup vs baseline: 1.4010x; 1.0060x over previous
import jax
import jax.numpy as jnp
from jax.experimental import pallas as pl
from jax.experimental.pallas import tpu as pltpu

_TI = 256
_CC = 64
_HP = jax.lax.Precision.HIGHEST


def _lrelu(v):
    return jnp.where(v >= 0, v, 0.2 * v)


def _proj_body(x_ref, wlT_ref, bl_ref, attv_ref, hl_ref, hlsT_ref,
               alrow_ref):
    x = x_ref[...]
    hl = jnp.dot(x, wlT_ref[...], precision=_HP) + bl_ref[...]
    hl_ref[...] = hl
    hlsT = (hl * attv_ref[...]).T
    hlsT_ref[...] = hlsT
    alrow_ref[...] = jnp.sum(hlsT, axis=0, keepdims=True)


def _proj(x, wlT, bl, att):
    n, cin = x.shape
    cout = wlT.shape[1]
    return pl.pallas_call(
        _proj_body,
        in_specs=[
            pl.BlockSpec((n, cin), lambda: (0, 0)),
            pl.BlockSpec((cin, cout), lambda: (0, 0)),
            pl.BlockSpec((1, cout), lambda: (0, 0)),
            pl.BlockSpec((1, cout), lambda: (0, 0)),
        ],
        out_specs=[
            pl.BlockSpec((n, cout), lambda: (0, 0)),
            pl.BlockSpec((cout, n), lambda: (0, 0)),
            pl.BlockSpec((1, n), lambda: (0, 0)),
        ],
        out_shape=[
            jax.ShapeDtypeStruct((n, cout), jnp.float32),
            jax.ShapeDtypeStruct((cout, n), jnp.float32),
            jax.ShapeDtypeStruct((1, n), jnp.float32),
        ],
    )(x, wlT, bl, att)


def _gat_body(x_ref, maskT_ref, wrT_ref, br_ref, attv_ref,
              atts_ref, bias_ref, hl_ref, hlsT_ref, alrow_ref,
              out_ref, s_ref):
    it = pl.program_id(0)
    n = maskT_ref.shape[1]
    cout = wrT_ref.shape[1]
    attv = attv_ref[...]
    x_t = x_ref[...]
    hr_t = jnp.dot(x_t, wrT_ref[...], precision=_HP) + br_ref[...]
    hl_t = hl_ref[pl.ds(it * _TI, _TI), :]
    hrs_t = hr_t * attv

    ar = jnp.sum(hrs_t, axis=1, keepdims=True)
    s_ref[...] = 0.6 * (ar + alrow_ref[...])

    for k in range(cout // _CC):
        acc = None
        for u in range(_CC):
            c = k * _CC + u
            a = atts_ref[0, c]
            g = jnp.where(a >= 0, jnp.float32(0.4), jnp.float32(-0.4))
            col = hrs_t[:, c:c + 1]
            row = hlsT_ref[c:c + 1, :]
            term = g * jnp.abs(col + row)
            acc = term if acc is None else acc + term
        s_ref[...] += acc

    Sm = s_ref[...] + maskT_ref[...]
    d = jnp.sum(_lrelu(hr_t + hl_t) * attv, axis=1, keepdims=True)
    mx = jnp.max(Sm, axis=1, keepdims=True)
    mx = jnp.maximum(mx, d)
    P = jnp.exp(Sm - mx)
    p_self = jnp.exp(d - mx)
    denom = jnp.sum(P, axis=1, keepdims=True) + p_self + 1e-16
    num = jnp.dot(P, hl_ref[...],
                  precision=jax.lax.Precision.DEFAULT) + p_self * hl_t
    out = num / denom + bias_ref[...]
    out_ref[...] = jnp.maximum(out, 0.0)


def _gat_layer(x, maskT, p):
    n, cin = x.shape
    cout = p["Wl"].shape[0]
    wlT = p["Wl"].T
    wrT = p["Wr"].T
    bl = p["bl"].reshape(1, cout)
    br = p["br"].reshape(1, cout)
    att = p["att"].reshape(1, cout)
    bias = p["bias"].reshape(1, cout)
    hl, hlsT, alrow = _proj(x, wlT, bl, att)
    return pl.pallas_call(
        _gat_body,
        grid=(n // _TI,),
        in_specs=[
            pl.BlockSpec((_TI, cin), lambda i: (i, 0)),
            pl.BlockSpec((_TI, n), lambda i: (i, 0)),
            pl.BlockSpec((cin, cout), lambda i: (0, 0)),
            pl.BlockSpec((1, cout), lambda i: (0, 0)),
            pl.BlockSpec((1, cout), lambda i: (0, 0)),
            pl.BlockSpec(memory_space=pltpu.SMEM),
            pl.BlockSpec((1, cout), lambda i: (0, 0)),
            pl.BlockSpec((n, cout), lambda i: (0, 0)),
            pl.BlockSpec((cout, n), lambda i: (0, 0)),
            pl.BlockSpec((1, n), lambda i: (0, 0)),
        ],
        out_specs=pl.BlockSpec((_TI, cout), lambda i: (i, 0)),
        out_shape=jax.ShapeDtypeStruct((n, cout), jnp.float32),
        scratch_shapes=[pltpu.VMEM((_TI, n), jnp.float32)],
        compiler_params=pltpu.CompilerParams(
            dimension_semantics=("parallel",)),
    )(x, maskT, wrT, br, att, att, bias, hl, hlsT, alrow)


def _recon_body(re_ref, out_ref):
    it = pl.program_id(0)
    re = re_ref[...]
    re_t = re_ref[pl.ds(it * _TI, _TI), :]
    logits = jnp.dot(re_t, re.T, precision=_HP)
    out_ref[...] = jax.nn.sigmoid(logits)


def _recon(re):
    n, c = re.shape
    return pl.pallas_call(
        _recon_body,
        grid=(n // _TI,),
        in_specs=[pl.BlockSpec((n, c), lambda i: (0, 0))],
        out_specs=pl.BlockSpec((_TI, n), lambda i: (i, 0)),
        out_shape=jax.ShapeDtypeStruct((n, n), jnp.float32),
        compiler_params=pltpu.CompilerParams(
            dimension_semantics=("parallel",)),
    )(re)


def kernel(x, edge_index, params):
    maskT = (edge_index.T == 0).astype(jnp.float32) * (-1e5)
    x1 = _gat_layer(x, maskT, params["conv1"])
    z = _gat_layer(x1, maskT, params["conv2"])
    re = _gat_layer(z, maskT, params["edge_dec"])
    recon = _recon(re)
    xd = _gat_layer(z, maskT, params["x_dec1"])
    xr = _gat_layer(xd, maskT, params["x_dec2"])
    return recon, xr, z
